# baseline (speedup 1.0000x reference)
import functools

import jax
import jax.numpy as jnp
from jax import lax
from jax.experimental import pallas as pl
from jax.experimental.pallas import tpu as pltpu
from jax.experimental.pallas import tpu_sc as plsc

_N = 10000
_E = 320000
_D = 128
_B = 16

_NC = 2
_NS = 16
_NW = _NC * _NS

_K = 112
_NCH = -(-_E // (_NW * _K))
_EPT = _NCH * _K
_EPAD = _NW * _EPT
_XPAD = 8
_ZR = 80
_NZ = _N // _ZR
_ZPT = -(-_NZ // _NS)

@functools.cache
def _make_sc_scatter():
    mesh = plsc.VectorSubcoreMesh(core_axis_name="c", subcore_axis_name="s")
    return functools.partial(
        pl.kernel,
        mesh=mesh,
        out_type=jax.ShapeDtypeStruct((_NC * _N, _D), jnp.float32),
        scratch_types=[
            pltpu.VMEM((_EPT,), jnp.int32),
            pltpu.VMEM((_NCH, _K), jnp.int32),
            pltpu.VMEM((_K, _D), jnp.float32),
            pltpu.VMEM((_K, _D), jnp.float32),
            pltpu.VMEM_SHARED((_N, _D), jnp.float32),
            pltpu.SemaphoreType.DMA,
            pltpu.SemaphoreType.DMA,
        ],
    )(_sc_scatter_body)


def _sc_scatter_body(x_hbm, src_hbm, dst_hbm, out_hbm, src_v, dst_v,
                     rows0, rows1, agg_sh, sem0, sem1):
    c = lax.axis_index("c")
    s = lax.axis_index("s")
    w = s * _NC + c

    pltpu.sync_copy(src_hbm.at[w], src_v)
    pltpu.sync_copy(dst_hbm.at[w], dst_v)

    def _zrow(r, carry):
        for cc in range(_D // 16):
            rows0[r, pl.ds(cc * 16, 16)] = jnp.zeros((16,), jnp.float32)
        return carry

    lax.fori_loop(0, _K, _zrow, 0)

    def _zchunk(i, carry):
        z = s + i * _NS

        @pl.when(z < _NZ)
        def _():
            pltpu.sync_copy(rows0.at[pl.ds(0, _ZR)],
                            agg_sh.at[pl.ds(z * _ZR, _ZR)])

        return carry

    lax.fori_loop(0, _ZPT, _zchunk, 0)

    pltpu.async_copy(x_hbm.at[src_v.at[pl.ds(0, _K)]], rows0, sem0)
    pltpu.async_copy(x_hbm.at[src_v.at[pl.ds(_K, _K)]], rows1, sem1)
    plsc.subcore_barrier()

    bufs = (rows0, rows1)
    sems = (sem0, sem1)

    def _edge2(t, carry):
        for b in range(2):
            j = 2 * t + b

            @pl.when(j < _NCH)
            def _():
                pltpu.make_async_copy(x_hbm.at[src_v.at[pl.ds(j * _K, _K)]],
                                      bufs[b], sems[b]).wait()
                pltpu.sync_copy(bufs[b], agg_sh.at[dst_v.at[j]], add=True)

                @pl.when(j + 2 < _NCH)
                def _():
                    pltpu.async_copy(
                        x_hbm.at[src_v.at[pl.ds((j + 2) * _K, _K)]],
                        bufs[b], sems[b])

        return carry

    lax.fori_loop(0, (_NCH + 1) // 2, _edge2, 0)
    plsc.subcore_barrier()

    for i in range(_ZPT):
        z = s + i * _NS
        b = i % 2

        @pl.when(z < _NZ)
        def _(i=i, z=z, b=b):
            stage = bufs[b].at[pl.ds(0, _ZR)]
            if i >= 2:
                pltpu.make_async_copy(
                    stage, out_hbm.at[pl.ds(c * _N + (z - 2 * _NS) * _ZR, _ZR)],
                    sems[b]).wait()
            pltpu.sync_copy(agg_sh.at[pl.ds(z * _ZR, _ZR)], stage)
            pltpu.async_copy(stage, out_hbm.at[pl.ds(c * _N + z * _ZR, _ZR)],
                             sems[b])

    for i in range(max(_ZPT - 2, 0), _ZPT):
        z = s + i * _NS
        b = i % 2

        @pl.when(z < _NZ)
        def _(z=z, b=b):
            pltpu.make_async_copy(
                bufs[b].at[pl.ds(0, _ZR)],
                out_hbm.at[pl.ds(c * _N + z * _ZR, _ZR)],
                sems[b]).wait()


_RB = 1000
_NBLK = _N // _RB


def _ln(h, g, b):
    m = jnp.mean(h, axis=-1, keepdims=True)
    v = jnp.mean((h - m) ** 2, axis=-1, keepdims=True)
    return (h - m) / jnp.sqrt(v + 1e-5) * g + b


def _mlp_block(x_ref, a0_ref, a1_ref, W1_ref, b1_ref, g1_ref, be1_ref,
               W2_ref, b2_ref, g2_ref, be2_ref):
    h = x_ref[...] + a0_ref[...] + a1_ref[...]
    h = jnp.dot(h, W1_ref[...], preferred_element_type=jnp.float32) + b1_ref[...]
    h = _ln(h, g1_ref[...], be1_ref[...])
    h = jnp.dot(h, W2_ref[...], preferred_element_type=jnp.float32) + b2_ref[...]
    h = _ln(h, g2_ref[...], be2_ref[...])
    return jnp.maximum(h, 0.0)


def _mlp_kernel(x_ref, a0_ref, a1_ref, W1_ref, b1_ref, g1_ref, be1_ref,
                W2_ref, b2_ref, g2_ref, be2_ref, o_ref):
    o_ref[...] = _mlp_block(x_ref, a0_ref, a1_ref, W1_ref, b1_ref, g1_ref,
                            be1_ref, W2_ref, b2_ref, g2_ref, be2_ref)


def _mlp_pool_kernel(x_ref, a0_ref, a1_ref, W1_ref, b1_ref, g1_ref, be1_ref,
                     W2_ref, b2_ref, g2_ref, be2_ref, batch_ref, Wl_ref,
                     bl_ref, o_ref, acc_s, acc_c):
    i = pl.program_id(0)
    h = _mlp_block(x_ref, a0_ref, a1_ref, W1_ref, b1_ref, g1_ref, be1_ref,
                   W2_ref, b2_ref, g2_ref, be2_ref)

    bvals = batch_ref[0]
    gid = lax.broadcasted_iota(jnp.int32, (_B, _RB), 0)
    oh = (gid == bvals).astype(jnp.float32)
    part_s = jnp.dot(oh, h, preferred_element_type=jnp.float32)
    part_c = jnp.sum(oh, axis=-1, keepdims=True)

    @pl.when(i == 0)
    def _():
        acc_s[...] = jnp.zeros_like(acc_s)
        acc_c[...] = jnp.zeros_like(acc_c)

    acc_s[...] += part_s
    acc_c[...] = acc_c[...] + part_c

    @pl.when(i == _NBLK - 1)
    def _():
        pooled = acc_s[...] / jnp.maximum(acc_c[...], 1.0)
        o_ref[...] = (jnp.dot(pooled, Wl_ref[...],
                              preferred_element_type=jnp.float32) + bl_ref[...])


def _row_spec(off_blocks=0):
    return pl.BlockSpec((_RB, _D), lambda i, o=off_blocks: (i + o, 0))


def _full_spec(shape):
    return pl.BlockSpec(shape, lambda i: tuple(0 for _ in shape))


_W_SPECS = [
    _full_spec((_D, _D)), _full_spec((1, _D)), _full_spec((1, _D)),
    _full_spec((1, _D)),
    _full_spec((_D, _D)), _full_spec((1, _D)), _full_spec((1, _D)),
    _full_spec((1, _D)),
]

_mlp_call = pl.pallas_call(
    _mlp_kernel,
    grid=(_NBLK,),
    in_specs=[_row_spec(), _row_spec(), _row_spec(_NBLK)] + _W_SPECS,
    out_specs=_row_spec(),
    out_shape=jax.ShapeDtypeStruct((_N, _D), jnp.float32),
    compiler_params=pltpu.CompilerParams(
        dimension_semantics=("arbitrary",)),
)

_mlp_pool_call = pl.pallas_call(
    _mlp_pool_kernel,
    grid=(_NBLK,),
    in_specs=[_row_spec(), _row_spec(), _row_spec(_NBLK)] + _W_SPECS + [
        pl.BlockSpec((1, 1, _RB), lambda i: (i, 0, 0)),
        _full_spec((_D, _D)),
        _full_spec((1, _D)),
    ],
    out_specs=_full_spec((_B, _D)),
    out_shape=jax.ShapeDtypeStruct((_B, _D), jnp.float32),
    scratch_shapes=[
        pltpu.VMEM((_B, _D), jnp.float32),
        pltpu.VMEM((_B, 1), jnp.float32),
    ],
    compiler_params=pltpu.CompilerParams(
        dimension_semantics=("arbitrary",)),
)


def kernel(x, edge_index, batch, W1a, b1a, g1a, be1a, W2a, b2a, g2a, be2a,
           W1b, b1b, g1b, be1b, W2b, b2b, g2b, be2b, Wl, bl):
    npad = _EPAD - _E
    src = jnp.concatenate(
        [edge_index[0], jnp.full((npad,), _N, jnp.int32)]).reshape(_NW, _EPT)
    dst = jnp.concatenate(
        [edge_index[1], jnp.zeros((npad,), jnp.int32)]).reshape(_NW, _NCH, _K)
    batch3 = batch.reshape(_NBLK, 1, _RB)

    r = lambda v: v.reshape(1, _D)
    pad_rows = lambda t: jnp.concatenate(
        [t, jnp.zeros((_XPAD, _D), jnp.float32)])

    sc_scatter = _make_sc_scatter()
    agg1 = sc_scatter(pad_rows(x), src, dst)
    h1 = _mlp_call(x, agg1, agg1, W1a, r(b1a), r(g1a), r(be1a),
                   W2a, r(b2a), r(g2a), r(be2a))
    agg2 = sc_scatter(pad_rows(h1), src, dst)
    out = _mlp_pool_call(h1, agg2, agg2, W1b, r(b1b), r(g1b), r(be1b),
                         W2b, r(b2b), r(g2b), r(be2b), batch3, Wl, r(bl))
    return out


# speedup vs baseline: 1.3548x; 1.3548x over previous
import functools

import jax
import jax.numpy as jnp
from jax import lax
from jax.experimental import pallas as pl
from jax.experimental.pallas import tpu as pltpu
from jax.experimental.pallas import tpu_sc as plsc

_N = 10000
_E = 320000
_D = 128
_B = 16

_NC = 2
_NS = 16
_NW = _NC * _NS

_K = 80
_NCH = _E // (_NW * _K)
_NSLOT = 3
_ZR = 80
_NZ = _N // _ZR
_ZPT = -(-_NZ // _NS)

@functools.cache
def _make_sc_scatter():
    mesh = plsc.VectorSubcoreMesh(core_axis_name="c", subcore_axis_name="s")
    return functools.partial(
        pl.kernel,
        mesh=mesh,
        out_type=jax.ShapeDtypeStruct((_NC * _N, _D), jnp.float32),
        scratch_types=(
            [pltpu.VMEM((_K, _D), jnp.float32)] * _NSLOT
            + [pltpu.VMEM((_K,), jnp.int32)] * _NSLOT
            + [
                pltpu.VMEM((_NCH, _K), jnp.int32),
                pltpu.VMEM_SHARED((_N, _D), jnp.float32),
                pltpu.SemaphoreType.DMA((_NSLOT,)),
                pltpu.SemaphoreType.DMA((_NSLOT,)),
                pltpu.SemaphoreType.DMA((_NSLOT,)),
            ]
        ),
    )(_sc_scatter_body)


def _sc_scatter_body(x_hbm, src_hbm, dst_hbm, out_hbm,
                     rows0, rows1, rows2, sb0, sb1, sb2,
                     dst_v, agg_sh, sem_is, sem_g, sem_s):
    c = lax.axis_index("c")
    s = lax.axis_index("s")
    w = s * _NC + c
    rows = (rows0, rows1, rows2)
    srcb = (sb0, sb1, sb2)
    ept = _NCH * _K

    def src_load(j, q):
        return pltpu.make_async_copy(
            src_hbm.at[pl.ds(w * ept + j * _K, _K)], srcb[q], sem_is.at[q])

    def gather(q):
        return pltpu.make_async_copy(x_hbm.at[srcb[q]], rows[q], sem_g.at[q])

    def scatter(j, q):
        return pltpu.make_async_copy(rows[q], agg_sh.at[dst_v.at[j]],
                                     sem_s.at[q])

    pltpu.sync_copy(dst_hbm.at[w], dst_v)

    def _zrow(r, carry):
        for cc in range(_D // 16):
            rows0[r, pl.ds(cc * 16, 16)] = jnp.zeros((16,), jnp.float32)
        return carry

    lax.fori_loop(0, _K, _zrow, 0)

    def _zchunk(i, carry):
        z = s + i * _NS

        @pl.when(z < _NZ)
        def _():
            pltpu.sync_copy(rows0, agg_sh.at[pl.ds(z * _ZR, _ZR)])

        return carry

    lax.fori_loop(0, _ZPT, _zchunk, 0)

    for q in range(_NSLOT):
        src_load(q, q).start()
    src_load(0, 0).wait()
    gather(0).start()
    plsc.subcore_barrier()

    def _edge3(t, carry):
        for b in range(_NSLOT):
            j = _NSLOT * t + b
            gather(b).wait()
            scatter(j, b).start(add=True)

            @pl.when(j + _NSLOT < _NCH)
            def _():
                src_load(j + _NSLOT, b).start()

            p = (b + 1) % _NSLOT

            @pl.when(j + 1 < _NCH)
            def _(j=j, p=p):
                @pl.when(j >= 2)
                def _():
                    scatter(j - 2, p).wait()

                src_load(j + 1, p).wait()
                gather(p).start()

        return carry

    nfull = (_NCH - 2) // _NSLOT
    lax.fori_loop(0, nfull, _edge3, 0)

    for j in range(nfull * _NSLOT, _NCH):
        q = j % _NSLOT
        gather(q).wait()
        scatter(j, q).start(add=True)
        if j + 1 < _NCH:
            p = (j + 1) % _NSLOT
            scatter(j - 2, p).wait()
            src_load(j + 1, p).wait()
            gather(p).start()
    for j in range(_NCH - _NSLOT, _NCH):
        scatter(j, j % _NSLOT).wait()
    plsc.subcore_barrier()

    for i in range(_ZPT):
        z = s + i * _NS
        b = i % _NSLOT

        @pl.when(z < _NZ)
        def _(i=i, z=z, b=b):
            if i >= _NSLOT:
                pltpu.make_async_copy(
                    rows[b],
                    out_hbm.at[pl.ds(c * _N + (z - _NSLOT * _NS) * _ZR, _ZR)],
                    sem_g.at[b]).wait()
            pltpu.sync_copy(agg_sh.at[pl.ds(z * _ZR, _ZR)], rows[b])
            pltpu.async_copy(rows[b],
                             out_hbm.at[pl.ds(c * _N + z * _ZR, _ZR)],
                             sem_g.at[b])

    for i in range(max(_ZPT - _NSLOT, 0), _ZPT):
        z = s + i * _NS
        b = i % _NSLOT

        @pl.when(z < _NZ)
        def _(z=z, b=b):
            pltpu.make_async_copy(
                rows[b],
                out_hbm.at[pl.ds(c * _N + z * _ZR, _ZR)],
                sem_g.at[b]).wait()


_RB = 1000
_NBLK = _N // _RB


def _ln(h, g, b):
    m = jnp.mean(h, axis=-1, keepdims=True)
    v = jnp.mean((h - m) ** 2, axis=-1, keepdims=True)
    return (h - m) / jnp.sqrt(v + 1e-5) * g + b


def _mlp_block(x_ref, a0_ref, a1_ref, W1_ref, b1_ref, g1_ref, be1_ref,
               W2_ref, b2_ref, g2_ref, be2_ref):
    h = x_ref[...] + a0_ref[...] + a1_ref[...]
    h = jnp.dot(h, W1_ref[...], preferred_element_type=jnp.float32) + b1_ref[...]
    h = _ln(h, g1_ref[...], be1_ref[...])
    h = jnp.dot(h, W2_ref[...], preferred_element_type=jnp.float32) + b2_ref[...]
    h = _ln(h, g2_ref[...], be2_ref[...])
    return jnp.maximum(h, 0.0)


def _mlp_kernel(x_ref, a0_ref, a1_ref, W1_ref, b1_ref, g1_ref, be1_ref,
                W2_ref, b2_ref, g2_ref, be2_ref, o_ref):
    o_ref[...] = _mlp_block(x_ref, a0_ref, a1_ref, W1_ref, b1_ref, g1_ref,
                            be1_ref, W2_ref, b2_ref, g2_ref, be2_ref)


def _mlp_pool_kernel(x_ref, a0_ref, a1_ref, W1_ref, b1_ref, g1_ref, be1_ref,
                     W2_ref, b2_ref, g2_ref, be2_ref, batch_ref, Wl_ref,
                     bl_ref, o_ref, acc_s, acc_c):
    i = pl.program_id(0)
    h = _mlp_block(x_ref, a0_ref, a1_ref, W1_ref, b1_ref, g1_ref, be1_ref,
                   W2_ref, b2_ref, g2_ref, be2_ref)

    bvals = batch_ref[0]
    gid = lax.broadcasted_iota(jnp.int32, (_B, _RB), 0)
    oh = (gid == bvals).astype(jnp.float32)
    part_s = jnp.dot(oh, h, preferred_element_type=jnp.float32)
    part_c = jnp.sum(oh, axis=-1, keepdims=True)

    @pl.when(i == 0)
    def _():
        acc_s[...] = jnp.zeros_like(acc_s)
        acc_c[...] = jnp.zeros_like(acc_c)

    acc_s[...] += part_s
    acc_c[...] = acc_c[...] + part_c

    @pl.when(i == _NBLK - 1)
    def _():
        pooled = acc_s[...] / jnp.maximum(acc_c[...], 1.0)
        o_ref[...] = (jnp.dot(pooled, Wl_ref[...],
                              preferred_element_type=jnp.float32) + bl_ref[...])


def _row_spec(off_blocks=0):
    return pl.BlockSpec((_RB, _D), lambda i, o=off_blocks: (i + o, 0))


def _full_spec(shape):
    return pl.BlockSpec(shape, lambda i: tuple(0 for _ in shape))


_W_SPECS = [
    _full_spec((_D, _D)), _full_spec((1, _D)), _full_spec((1, _D)),
    _full_spec((1, _D)),
    _full_spec((_D, _D)), _full_spec((1, _D)), _full_spec((1, _D)),
    _full_spec((1, _D)),
]

_mlp_call = pl.pallas_call(
    _mlp_kernel,
    grid=(_NBLK,),
    in_specs=[_row_spec(), _row_spec(), _row_spec(_NBLK)] + _W_SPECS,
    out_specs=_row_spec(),
    out_shape=jax.ShapeDtypeStruct((_N, _D), jnp.float32),
    compiler_params=pltpu.CompilerParams(
        dimension_semantics=("arbitrary",)),
)

_mlp_pool_call = pl.pallas_call(
    _mlp_pool_kernel,
    grid=(_NBLK,),
    in_specs=[_row_spec(), _row_spec(), _row_spec(_NBLK)] + _W_SPECS + [
        pl.BlockSpec((1, 1, _RB), lambda i: (i, 0, 0)),
        _full_spec((_D, _D)),
        _full_spec((1, _D)),
    ],
    out_specs=_full_spec((_B, _D)),
    out_shape=jax.ShapeDtypeStruct((_B, _D), jnp.float32),
    scratch_shapes=[
        pltpu.VMEM((_B, _D), jnp.float32),
        pltpu.VMEM((_B, 1), jnp.float32),
    ],
    compiler_params=pltpu.CompilerParams(
        dimension_semantics=("arbitrary",)),
)


def kernel(x, edge_index, batch, W1a, b1a, g1a, be1a, W2a, b2a, g2a, be2a,
           W1b, b1b, g1b, be1b, W2b, b2b, g2b, be2b, Wl, bl):
    src = edge_index[0]
    dst = edge_index[1].reshape(_NW, _NCH, _K)
    batch3 = batch.reshape(_NBLK, 1, _RB)

    r = lambda v: v.reshape(1, _D)

    sc_scatter = _make_sc_scatter()
    agg1 = sc_scatter(x, src, dst)
    h1 = _mlp_call(x, agg1, agg1, W1a, r(b1a), r(g1a), r(be1a),
                   W2a, r(b2a), r(g2a), r(be2a))
    agg2 = sc_scatter(h1, src, dst)
    out = _mlp_pool_call(h1, agg2, agg2, W1b, r(b1b), r(g1b), r(be1b),
                         W2b, r(b2b), r(g2b), r(be2b), batch3, Wl, r(bl))
    return out


# speedup vs baseline: 1.7142x; 1.2652x over previous
import functools

import jax
import jax.numpy as jnp
from jax import lax
from jax.experimental import pallas as pl
from jax.experimental.pallas import tpu as pltpu
from jax.experimental.pallas import tpu_sc as plsc

_N = 10000
_E = 320000
_D = 128
_B = 16

_NC = 2
_NS = 16
_NW = _NC * _NS

_K = 80
_NCH = _E // (_NW * _K)
_NSLOT = 2
_ZR = 80
_NZ = _N // _ZR
_ZPT = -(-_NZ // _NS)

@functools.cache
def _make_sc_scatter():
    mesh = plsc.VectorSubcoreMesh(core_axis_name="c", subcore_axis_name="s")
    return functools.partial(
        pl.kernel,
        mesh=mesh,
        out_type=jax.ShapeDtypeStruct((_NC * _N, _D), jnp.float32),
        scratch_types=(
            [pltpu.VMEM((_K, _D), jnp.float32)] * _NSLOT
            + [
                pltpu.VMEM((_NCH * _K,), jnp.int32),
                pltpu.VMEM((_NCH, _K), jnp.int32),
                pltpu.VMEM_SHARED((_N, _D), jnp.float32),
                pltpu.SemaphoreType.DMA((_NSLOT,)),
            ]
        ),
    )(_sc_scatter_body)


def _sc_scatter_body(x_hbm, src_hbm, dst_hbm, out_hbm,
                     rows0, rows1, src_v, dst_v, agg_sh, sem_g):
    c = lax.axis_index("c")
    s = lax.axis_index("s")
    w = s * _NC + c
    rows = (rows0, rows1)
    ept = _NCH * _K

    def gather(j, q):
        return pltpu.make_async_copy(
            x_hbm.at[src_v.at[pl.ds(j * _K, _K)]], rows[q], sem_g.at[q])

    pltpu.sync_copy(src_hbm.at[pl.ds(w * ept, ept)], src_v)
    pltpu.sync_copy(dst_hbm.at[w], dst_v)

    def _zrow(r, carry):
        for cc in range(_D // 16):
            rows0[r, pl.ds(cc * 16, 16)] = jnp.zeros((16,), jnp.float32)
        return carry

    lax.fori_loop(0, _K, _zrow, 0)

    def _zchunk(i, carry):
        z = s + i * _NS

        @pl.when(z < _NZ)
        def _():
            pltpu.sync_copy(rows0, agg_sh.at[pl.ds(z * _ZR, _ZR)])

        return carry

    lax.fori_loop(0, _ZPT, _zchunk, 0)

    gather(0, 0).start()
    gather(1, 1).start()
    plsc.subcore_barrier()

    def _edge2(t, carry):
        for b in range(_NSLOT):
            j = _NSLOT * t + b

            @pl.when(j < _NCH)
            def _(j=j, b=b):
                gather(j, b).wait()
                pltpu.sync_copy(rows[b], agg_sh.at[dst_v.at[j]], add=True)

                @pl.when(j + _NSLOT < _NCH)
                def _():
                    gather(j + _NSLOT, b).start()

        return carry

    lax.fori_loop(0, (_NCH + 1) // _NSLOT, _edge2, 0)
    plsc.subcore_barrier()

    for i in range(_ZPT):
        z = s + i * _NS
        b = i % _NSLOT

        @pl.when(z < _NZ)
        def _(i=i, z=z, b=b):
            if i >= _NSLOT:
                pltpu.make_async_copy(
                    rows[b],
                    out_hbm.at[pl.ds(c * _N + (z - _NSLOT * _NS) * _ZR, _ZR)],
                    sem_g.at[b]).wait()
            pltpu.sync_copy(agg_sh.at[pl.ds(z * _ZR, _ZR)], rows[b])
            pltpu.async_copy(rows[b],
                             out_hbm.at[pl.ds(c * _N + z * _ZR, _ZR)],
                             sem_g.at[b])

    for i in range(max(_ZPT - _NSLOT, 0), _ZPT):
        z = s + i * _NS
        b = i % _NSLOT

        @pl.when(z < _NZ)
        def _(z=z, b=b):
            pltpu.make_async_copy(
                rows[b],
                out_hbm.at[pl.ds(c * _N + z * _ZR, _ZR)],
                sem_g.at[b]).wait()


_RB = 1000
_NBLK = _N // _RB


def _ln(h, g, b):
    m = jnp.mean(h, axis=-1, keepdims=True)
    v = jnp.mean((h - m) ** 2, axis=-1, keepdims=True)
    return (h - m) / jnp.sqrt(v + 1e-5) * g + b


def _mlp_block(x_ref, a0_ref, a1_ref, W1_ref, b1_ref, g1_ref, be1_ref,
               W2_ref, b2_ref, g2_ref, be2_ref):
    h = x_ref[...] + a0_ref[...] + a1_ref[...]
    h = jnp.dot(h, W1_ref[...], preferred_element_type=jnp.float32) + b1_ref[...]
    h = _ln(h, g1_ref[...], be1_ref[...])
    h = jnp.dot(h, W2_ref[...], preferred_element_type=jnp.float32) + b2_ref[...]
    h = _ln(h, g2_ref[...], be2_ref[...])
    return jnp.maximum(h, 0.0)


def _mlp_kernel(x_ref, a0_ref, a1_ref, W1_ref, b1_ref, g1_ref, be1_ref,
                W2_ref, b2_ref, g2_ref, be2_ref, o_ref):
    o_ref[...] = _mlp_block(x_ref, a0_ref, a1_ref, W1_ref, b1_ref, g1_ref,
                            be1_ref, W2_ref, b2_ref, g2_ref, be2_ref)


def _mlp_pool_kernel(x_ref, a0_ref, a1_ref, W1_ref, b1_ref, g1_ref, be1_ref,
                     W2_ref, b2_ref, g2_ref, be2_ref, batch_ref, Wl_ref,
                     bl_ref, o_ref, acc_s, acc_c):
    i = pl.program_id(0)
    h = _mlp_block(x_ref, a0_ref, a1_ref, W1_ref, b1_ref, g1_ref, be1_ref,
                   W2_ref, b2_ref, g2_ref, be2_ref)

    bvals = batch_ref[0]
    gid = lax.broadcasted_iota(jnp.int32, (_B, _RB), 0)
    oh = (gid == bvals).astype(jnp.float32)
    part_s = jnp.dot(oh, h, preferred_element_type=jnp.float32)
    part_c = jnp.sum(oh, axis=-1, keepdims=True)

    @pl.when(i == 0)
    def _():
        acc_s[...] = jnp.zeros_like(acc_s)
        acc_c[...] = jnp.zeros_like(acc_c)

    acc_s[...] += part_s
    acc_c[...] = acc_c[...] + part_c

    @pl.when(i == _NBLK - 1)
    def _():
        pooled = acc_s[...] / jnp.maximum(acc_c[...], 1.0)
        o_ref[...] = (jnp.dot(pooled, Wl_ref[...],
                              preferred_element_type=jnp.float32) + bl_ref[...])


def _row_spec(off_blocks=0):
    return pl.BlockSpec((_RB, _D), lambda i, o=off_blocks: (i + o, 0))


def _full_spec(shape):
    return pl.BlockSpec(shape, lambda i: tuple(0 for _ in shape))


_W_SPECS = [
    _full_spec((_D, _D)), _full_spec((1, _D)), _full_spec((1, _D)),
    _full_spec((1, _D)),
    _full_spec((_D, _D)), _full_spec((1, _D)), _full_spec((1, _D)),
    _full_spec((1, _D)),
]

_mlp_call = pl.pallas_call(
    _mlp_kernel,
    grid=(_NBLK,),
    in_specs=[_row_spec(), _row_spec(), _row_spec(_NBLK)] + _W_SPECS,
    out_specs=_row_spec(),
    out_shape=jax.ShapeDtypeStruct((_N, _D), jnp.float32),
    compiler_params=pltpu.CompilerParams(
        dimension_semantics=("arbitrary",)),
)

_mlp_pool_call = pl.pallas_call(
    _mlp_pool_kernel,
    grid=(_NBLK,),
    in_specs=[_row_spec(), _row_spec(), _row_spec(_NBLK)] + _W_SPECS + [
        pl.BlockSpec((1, 1, _RB), lambda i: (i, 0, 0)),
        _full_spec((_D, _D)),
        _full_spec((1, _D)),
    ],
    out_specs=_full_spec((_B, _D)),
    out_shape=jax.ShapeDtypeStruct((_B, _D), jnp.float32),
    scratch_shapes=[
        pltpu.VMEM((_B, _D), jnp.float32),
        pltpu.VMEM((_B, 1), jnp.float32),
    ],
    compiler_params=pltpu.CompilerParams(
        dimension_semantics=("arbitrary",)),
)


def kernel(x, edge_index, batch, W1a, b1a, g1a, be1a, W2a, b2a, g2a, be2a,
           W1b, b1b, g1b, be1b, W2b, b2b, g2b, be2b, Wl, bl):
    src = edge_index[0]
    dst = edge_index[1].reshape(_NW, _NCH, _K)
    batch3 = batch.reshape(_NBLK, 1, _RB)

    r = lambda v: v.reshape(1, _D)

    sc_scatter = _make_sc_scatter()
    agg1 = sc_scatter(x, src, dst)
    h1 = _mlp_call(x, agg1, agg1, W1a, r(b1a), r(g1a), r(be1a),
                   W2a, r(b2a), r(g2a), r(be2a))
    agg2 = sc_scatter(h1, src, dst)
    out = _mlp_pool_call(h1, agg2, agg2, W1b, r(b1b), r(g1b), r(be1b),
                         W2b, r(b2b), r(g2b), r(be2b), batch3, Wl, r(bl))
    return out


# speedup vs baseline: 1.7389x; 1.0144x over previous
import functools

import jax
import jax.numpy as jnp
from jax import lax
from jax.experimental import pallas as pl
from jax.experimental.pallas import tpu as pltpu
from jax.experimental.pallas import tpu_sc as plsc

_N = 10000
_E = 320000
_D = 128
_B = 16

_NC = 2
_NS = 16
_NW = _NC * _NS

_K = 80
_NCH = _E // (_NW * _K)
_NSLOT = 2
_ZR = 80
_NZ = _N // _ZR
_ZPT = -(-_NZ // _NS)

@functools.cache
def _make_sc_scatter():
    mesh = plsc.VectorSubcoreMesh(core_axis_name="c", subcore_axis_name="s")
    return functools.partial(
        pl.kernel,
        mesh=mesh,
        out_type=jax.ShapeDtypeStruct((_NC * _N, _D), jnp.float32),
        scratch_types=(
            [pltpu.VMEM((_K, _D), jnp.float32)] * _NSLOT
            + [
                pltpu.VMEM((_NCH * _K,), jnp.int32),
                pltpu.VMEM((_NCH, _K), jnp.int32),
                pltpu.VMEM_SHARED((_N, _D), jnp.float32),
                pltpu.SemaphoreType.DMA((_NSLOT,)),
            ]
        ),
    )(_sc_scatter_body)


def _sc_scatter_body(x_hbm, src_hbm, dst_hbm, out_hbm,
                     rows0, rows1, src_v, dst_v, agg_sh, sem_g):
    c = lax.axis_index("c")
    s = lax.axis_index("s")
    w = s * _NC + c
    rows = (rows0, rows1)
    ept = _NCH * _K

    def gather(j, q):
        return pltpu.make_async_copy(
            x_hbm.at[src_v.at[pl.ds(j * _K, _K)]], rows[q], sem_g.at[q])

    pltpu.sync_copy(src_hbm.at[pl.ds(w * ept, ept)], src_v)
    pltpu.sync_copy(dst_hbm.at[w], dst_v)

    def _zrow(r, carry):
        for cc in range(_D // 16):
            rows0[r, pl.ds(cc * 16, 16)] = jnp.zeros((16,), jnp.float32)
        return carry

    lax.fori_loop(0, _K, _zrow, 0)

    def _zchunk(i, carry):
        z = s + i * _NS

        @pl.when(z < _NZ)
        def _():
            pltpu.sync_copy(rows0, agg_sh.at[pl.ds(z * _ZR, _ZR)])

        return carry

    lax.fori_loop(0, _ZPT, _zchunk, 0)

    gather(0, 0).start()
    gather(1, 1).start()
    plsc.subcore_barrier()

    def _edge2(t, carry):
        for b in range(_NSLOT):
            j = _NSLOT * t + b

            @pl.when(j < _NCH)
            def _(j=j, b=b):
                gather(j, b).wait()
                pltpu.sync_copy(rows[b], agg_sh.at[dst_v.at[j]], add=True)

                @pl.when(j + _NSLOT < _NCH)
                def _():
                    gather(j + _NSLOT, b).start()

        return carry

    lax.fori_loop(0, (_NCH + 1) // _NSLOT, _edge2, 0)
    plsc.subcore_barrier()

    for i in range(_ZPT):
        z = s + i * _NS
        b = i % _NSLOT

        @pl.when(z < _NZ)
        def _(i=i, z=z, b=b):
            if i >= _NSLOT:
                pltpu.make_async_copy(
                    rows[b],
                    out_hbm.at[pl.ds(c * _N + (z - _NSLOT * _NS) * _ZR, _ZR)],
                    sem_g.at[b]).wait()
            pltpu.sync_copy(agg_sh.at[pl.ds(z * _ZR, _ZR)], rows[b])
            pltpu.async_copy(rows[b],
                             out_hbm.at[pl.ds(c * _N + z * _ZR, _ZR)],
                             sem_g.at[b])

    for i in range(max(_ZPT - _NSLOT, 0), _ZPT):
        z = s + i * _NS
        b = i % _NSLOT

        @pl.when(z < _NZ)
        def _(z=z, b=b):
            pltpu.make_async_copy(
                rows[b],
                out_hbm.at[pl.ds(c * _N + z * _ZR, _ZR)],
                sem_g.at[b]).wait()


_RB = 2000
_NBLK = _N // _RB


def _ln(h, g, b):
    m = jnp.mean(h, axis=-1, keepdims=True)
    v = jnp.mean((h - m) ** 2, axis=-1, keepdims=True)
    return (h - m) / jnp.sqrt(v + 1e-5) * g + b


def _mlp_block(x_ref, a0_ref, a1_ref, W1_ref, b1_ref, g1_ref, be1_ref,
               W2_ref, b2_ref, g2_ref, be2_ref):
    h = x_ref[...] + a0_ref[...] + a1_ref[...]
    h = jnp.dot(h, W1_ref[...], preferred_element_type=jnp.float32) + b1_ref[...]
    h = _ln(h, g1_ref[...], be1_ref[...])
    h = jnp.dot(h, W2_ref[...], preferred_element_type=jnp.float32) + b2_ref[...]
    h = _ln(h, g2_ref[...], be2_ref[...])
    return jnp.maximum(h, 0.0)


def _mlp_kernel(x_ref, a0_ref, a1_ref, W1_ref, b1_ref, g1_ref, be1_ref,
                W2_ref, b2_ref, g2_ref, be2_ref, o_ref):
    o_ref[...] = _mlp_block(x_ref, a0_ref, a1_ref, W1_ref, b1_ref, g1_ref,
                            be1_ref, W2_ref, b2_ref, g2_ref, be2_ref)


def _mlp_pool_kernel(x_ref, a0_ref, a1_ref, W1_ref, b1_ref, g1_ref, be1_ref,
                     W2_ref, b2_ref, g2_ref, be2_ref, batch_ref, Wl_ref,
                     bl_ref, o_ref, acc_s, acc_c):
    i = pl.program_id(0)
    h = _mlp_block(x_ref, a0_ref, a1_ref, W1_ref, b1_ref, g1_ref, be1_ref,
                   W2_ref, b2_ref, g2_ref, be2_ref)

    bvals = batch_ref[0]
    gid = lax.broadcasted_iota(jnp.int32, (_B, _RB), 0)
    oh = (gid == bvals).astype(jnp.float32)
    part_s = jnp.dot(oh, h, preferred_element_type=jnp.float32)
    part_c = jnp.sum(oh, axis=-1, keepdims=True)

    @pl.when(i == 0)
    def _():
        acc_s[...] = jnp.zeros_like(acc_s)
        acc_c[...] = jnp.zeros_like(acc_c)

    acc_s[...] += part_s
    acc_c[...] = acc_c[...] + part_c

    @pl.when(i == _NBLK - 1)
    def _():
        pooled = acc_s[...] / jnp.maximum(acc_c[...], 1.0)
        o_ref[...] = (jnp.dot(pooled, Wl_ref[...],
                              preferred_element_type=jnp.float32) + bl_ref[...])


def _row_spec(off_blocks=0):
    return pl.BlockSpec((_RB, _D), lambda i, o=off_blocks: (i + o, 0))


def _full_spec(shape):
    return pl.BlockSpec(shape, lambda i: tuple(0 for _ in shape))


_W_SPECS = [
    _full_spec((_D, _D)), _full_spec((1, _D)), _full_spec((1, _D)),
    _full_spec((1, _D)),
    _full_spec((_D, _D)), _full_spec((1, _D)), _full_spec((1, _D)),
    _full_spec((1, _D)),
]

_mlp_call = pl.pallas_call(
    _mlp_kernel,
    grid=(_NBLK,),
    in_specs=[_row_spec(), _row_spec(), _row_spec(_NBLK)] + _W_SPECS,
    out_specs=_row_spec(),
    out_shape=jax.ShapeDtypeStruct((_N, _D), jnp.float32),
    compiler_params=pltpu.CompilerParams(
        dimension_semantics=("arbitrary",)),
)

_mlp_pool_call = pl.pallas_call(
    _mlp_pool_kernel,
    grid=(_NBLK,),
    in_specs=[_row_spec(), _row_spec(), _row_spec(_NBLK)] + _W_SPECS + [
        pl.BlockSpec((1, 1, _RB), lambda i: (i, 0, 0)),
        _full_spec((_D, _D)),
        _full_spec((1, _D)),
    ],
    out_specs=_full_spec((_B, _D)),
    out_shape=jax.ShapeDtypeStruct((_B, _D), jnp.float32),
    scratch_shapes=[
        pltpu.VMEM((_B, _D), jnp.float32),
        pltpu.VMEM((_B, 1), jnp.float32),
    ],
    compiler_params=pltpu.CompilerParams(
        dimension_semantics=("arbitrary",)),
)


def kernel(x, edge_index, batch, W1a, b1a, g1a, be1a, W2a, b2a, g2a, be2a,
           W1b, b1b, g1b, be1b, W2b, b2b, g2b, be2b, Wl, bl):
    src = edge_index[0]
    dst = edge_index[1].reshape(_NW, _NCH, _K)
    batch3 = batch.reshape(_NBLK, 1, _RB)

    r = lambda v: v.reshape(1, _D)

    sc_scatter = _make_sc_scatter()
    agg1 = sc_scatter(x, src, dst)
    h1 = _mlp_call(x, agg1, agg1, W1a, r(b1a), r(g1a), r(be1a),
                   W2a, r(b2a), r(g2a), r(be2a))
    agg2 = sc_scatter(h1, src, dst)
    out = _mlp_pool_call(h1, agg2, agg2, W1b, r(b1b), r(g1b), r(be1b),
                         W2b, r(b2b), r(g2b), r(be2b), batch3, Wl, r(bl))
    return out
